# Initial kernel scaffold; baseline (speedup 1.0000x reference)
import jax
import jax.numpy as jnp
from jax.experimental import pallas as pl


def kernel(x, edge_index, W0, b0, W1, b1, g0, bt0, g1, bt1):
    raise NotImplementedError("write your pallas kernel here")


# speedup vs baseline: 4.5664x; 4.5664x over previous
import functools

import jax
import jax.numpy as jnp
from jax import lax
from jax.experimental import pallas as pl
from jax.experimental.pallas import tpu as pltpu
from jax.experimental.pallas import tpu_sc as plsc

N_NODES = 10000
N_EDGES = 320000
D = 128
EPS = 1e-5

NC = 2
NS = 16
NW = NC * NS
EPW = N_EDGES // NW
CHUNK = 100
NCH = EPW // CHUNK
NV = D // 16

_mesh = plsc.VectorSubcoreMesh(core_axis_name="c", subcore_axis_name="s")


def _cols():
    lane = lax.iota(jnp.int32, 16)
    return [lane + 16 * f for f in range(NV)]


def _wid():
    return lax.axis_index("s") * NC + lax.axis_index("c")


def _hist_body(src_hbm, dst_hbm, out_hbm, idx_v, cnt_u, cnt_d):
    wid = _wid()
    base = wid * EPW
    zeros = jnp.zeros((16,), jnp.float32)

    def zero_body(i, _):
        cnt_u[pl.ds(i * 16, 16)] = zeros
        cnt_d[pl.ds(i * 16, 16)] = zeros
        return 0

    lax.fori_loop(0, N_NODES // 16, zero_body, 0)

    ones = jnp.ones((16,), jnp.float32)

    def count_into(cnt_ref):
        def body(i, _):
            ii = idx_v[pl.ds(i * 16, 16)]
            c, last = plsc.scan_count(ii)
            plsc.addupdate_scatter(cnt_ref, [ii], c.astype(jnp.float32),
                                   mask=last)
            return 0
        lax.fori_loop(0, EPW // 16, body, 0)

    pltpu.sync_copy(src_hbm.at[pl.ds(base, EPW)], idx_v)
    count_into(cnt_u)
    pltpu.sync_copy(dst_hbm.at[pl.ds(base, EPW)], idx_v)
    count_into(cnt_d)

    pltpu.sync_copy(cnt_u, out_hbm.at[wid, 0])
    pltpu.sync_copy(cnt_d, out_hbm.at[wid, 1])


_hist = pl.kernel(
    _hist_body,
    out_type=jax.ShapeDtypeStruct((NW, 2, N_NODES), jnp.float32),
    mesh=_mesh,
    compiler_params=pltpu.CompilerParams(needs_layout_passes=False),
    scratch_types=[
        pltpu.VMEM((EPW,), jnp.int32),
        pltpu.VMEM((N_NODES,), jnp.float32),
        pltpu.VMEM((N_NODES,), jnp.float32),
    ],
)


def _fold_body(x_ref, cntp_ref, w0_ref, b0_ref, g0_ref, bt0_ref,
               zu_ref, zv_ref):
    x = x_ref[...]
    cnt = jnp.sum(cntp_ref[...], axis=0)
    dn = (((1,), (0,)), ((), ()))
    s = lax.dot_general(cnt, x, dn, preferred_element_type=jnp.float32)
    q = lax.dot_general(cnt, x * x, dn, preferred_element_type=jnp.float32)
    mu = s / N_EDGES
    var = jnp.maximum(q / N_EDGES - mu * mu, 0.0)
    a0 = g0_ref[...].reshape(2, D) * lax.rsqrt(var + EPS)
    c0 = bt0_ref[...].reshape(2, D) - mu * a0
    w0 = w0_ref[...]
    w0u = w0[:, :D] * a0[0][None, :]
    w0v = w0[:, D:] * a0[1][None, :]
    dnv = (((1,), (0,)), ((), ()))
    b0eff = (b0_ref[...]
             + lax.dot_general(w0[:, :D], c0[0], dnv,
                               preferred_element_type=jnp.float32)
             + lax.dot_general(w0[:, D:], c0[1], dnv,
                               preferred_element_type=jnp.float32))
    dnt = (((1,), (1,)), ((), ()))
    zu_ref[...] = lax.dot_general(x, w0u, dnt,
                                  preferred_element_type=jnp.float32)
    zv_ref[...] = lax.dot_general(x, w0v, dnt,
                                  preferred_element_type=jnp.float32) \
        + b0eff[None, :]


_fold = pl.pallas_call(
    _fold_body,
    out_shape=(jax.ShapeDtypeStruct((N_NODES, D), jnp.float32),
               jax.ShapeDtypeStruct((N_NODES, D), jnp.float32)),
)


def _stage_idx(src4_hbm, dst4_hbm, idxs):
    wid = _wid()
    pltpu.sync_copy(src4_hbm.at[wid], idxs.at[0])
    pltpu.sync_copy(dst4_hbm.at[wid], idxs.at[1])


def _start(zu_hbm, zv_hbm, idxs, ubuf, vbuf, sems, b, ci):
    pltpu.async_copy(zu_hbm.at[idxs.at[0, ci]], ubuf.at[b], sems[0][b])
    pltpu.async_copy(zv_hbm.at[idxs.at[1, ci]], vbuf.at[b], sems[1][b])


def _wait(zu_hbm, zv_hbm, idxs, ubuf, vbuf, sems, b, ci):
    pltpu.make_async_copy(zu_hbm.at[idxs.at[0, ci]], ubuf.at[b],
                          sems[0][b]).wait()
    pltpu.make_async_copy(zv_hbm.at[idxs.at[1, ci]], vbuf.at[b],
                          sems[1][b]).wait()


def _pass1_body(zu_hbm, zv_hbm, src4_hbm, dst4_hbm, out_hbm,
                idxs, ubuf, vbuf, sbuf, su0, su1, sv0, sv1):
    wid = _wid()
    sems = ((su0, su1), (sv0, sv1))
    COLS = _cols()
    _stage_idx(src4_hbm, dst4_hbm, idxs)

    _start(zu_hbm, zv_hbm, idxs, ubuf, vbuf, sems, 0, 0)
    _start(zu_hbm, zv_hbm, idxs, ubuf, vbuf, sems, 1, 1)

    def make_inner(b):
        ub = ubuf.at[b]
        vb = vbuf.at[b]

        def edge_body(e, accs):
            out = list(accs)
            row = jnp.full((16,), e, jnp.int32)
            for f in range(NV):
                u = plsc.load_gather(ub, [row, COLS[f]])
                v = plsc.load_gather(vb, [row, COLS[f]])
                r = jnp.maximum(u + v, 0.0)
                out[f] = out[f] + r
                out[NV + f] = out[NV + f] + r * r
            return tuple(out)

        return edge_body

    inner = [make_inner(0), make_inner(1)]

    def outer(cio, accs):
        for b in range(2):
            ci = 2 * cio + b
            _wait(zu_hbm, zv_hbm, idxs, ubuf, vbuf, sems, b, ci)
            accs = lax.fori_loop(0, CHUNK, inner[b], accs)
            _start(zu_hbm, zv_hbm, idxs, ubuf, vbuf, sems, b, ci + 2)
        return accs

    accs = tuple(jnp.zeros((16,), jnp.float32) for _ in range(2 * NV))
    accs = lax.fori_loop(0, NCH // 2 - 1, outer, accs)
    for b in range(2):
        ci = NCH - 2 + b
        _wait(zu_hbm, zv_hbm, idxs, ubuf, vbuf, sems, b, ci)
        accs = lax.fori_loop(0, CHUNK, inner[b], accs)

    for f in range(NV):
        sbuf[0, pl.ds(16 * f, 16)] = accs[f]
        sbuf[1, pl.ds(16 * f, 16)] = accs[NV + f]
    pltpu.sync_copy(sbuf, out_hbm.at[wid])


_pass1 = pl.kernel(
    _pass1_body,
    out_type=jax.ShapeDtypeStruct((NW, 2, D), jnp.float32),
    mesh=_mesh,
    compiler_params=pltpu.CompilerParams(needs_layout_passes=False),
    scratch_types=[
        pltpu.VMEM((2, NCH, CHUNK), jnp.int32),
        pltpu.VMEM((2, CHUNK, D), jnp.float32),
        pltpu.VMEM((2, CHUNK, D), jnp.float32),
        pltpu.VMEM((2, D), jnp.float32),
        pltpu.SemaphoreType.DMA,
        pltpu.SemaphoreType.DMA,
        pltpu.SemaphoreType.DMA,
        pltpu.SemaphoreType.DMA,
    ],
)


def _pass2_body(zu_hbm, zv_hbm, src4_hbm, dst4_hbm, wc_hbm, out_hbm,
                idxs, ubuf, vbuf, wcv, obuf, su0, su1, sv0, sv1):
    wid = _wid()
    base = wid * EPW
    sems = ((su0, su1), (sv0, sv1))
    COLS = _cols()
    _stage_idx(src4_hbm, dst4_hbm, idxs)
    pltpu.sync_copy(wc_hbm, wcv)
    wv = [wcv[0, pl.ds(16 * f, 16)] for f in range(NV)]
    cvec = wcv[1, pl.ds(0, 16)]
    lane15 = lax.iota(jnp.int32, 16) == 15

    _start(zu_hbm, zv_hbm, idxs, ubuf, vbuf, sems, 0, 0)
    _start(zu_hbm, zv_hbm, idxs, ubuf, vbuf, sems, 1, 1)

    def make_inner(b):
        ub = ubuf.at[b]
        vb = vbuf.at[b]

        def edge_body(e, ci):
            acc = None
            row = jnp.full((16,), e, jnp.int32)
            for f in range(NV):
                u = plsc.load_gather(ub, [row, COLS[f]])
                v = plsc.load_gather(vb, [row, COLS[f]])
                r = jnp.maximum(u + v, 0.0)
                t = r * wv[f]
                acc = t if acc is None else acc + t
            cs = plsc.cumsum(acc) + cvec
            tgt = jnp.full((16,), ci * CHUNK + e, jnp.int32)
            plsc.store_scatter(obuf, [tgt], cs, mask=lane15)
            return ci

        return edge_body

    inner = [make_inner(0), make_inner(1)]

    def outer(cio, _):
        for b in range(2):
            ci = 2 * cio + b
            _wait(zu_hbm, zv_hbm, idxs, ubuf, vbuf, sems, b, ci)
            lax.fori_loop(0, CHUNK, inner[b], ci)
            _start(zu_hbm, zv_hbm, idxs, ubuf, vbuf, sems, b, ci + 2)
        return 0

    lax.fori_loop(0, NCH // 2 - 1, outer, 0)
    for b in range(2):
        ci = NCH - 2 + b
        _wait(zu_hbm, zv_hbm, idxs, ubuf, vbuf, sems, b, ci)
        lax.fori_loop(0, CHUNK, inner[b], ci)

    pltpu.sync_copy(obuf, out_hbm.at[pl.ds(base, EPW)])


_pass2 = pl.kernel(
    _pass2_body,
    out_type=jax.ShapeDtypeStruct((N_EDGES,), jnp.float32),
    mesh=_mesh,
    compiler_params=pltpu.CompilerParams(needs_layout_passes=False),
    scratch_types=[
        pltpu.VMEM((2, NCH, CHUNK), jnp.int32),
        pltpu.VMEM((2, CHUNK, D), jnp.float32),
        pltpu.VMEM((2, CHUNK, D), jnp.float32),
        pltpu.VMEM((2, D), jnp.float32),
        pltpu.VMEM((EPW,), jnp.float32),
        pltpu.SemaphoreType.DMA,
        pltpu.SemaphoreType.DMA,
        pltpu.SemaphoreType.DMA,
        pltpu.SemaphoreType.DMA,
    ],
)


def kernel(x, edge_index, W0, b0, W1, b1, g0, bt0, g1, bt1):
    src = edge_index[0]
    dst = edge_index[1]
    src4 = src.reshape(NW, NCH, CHUNK)
    dst4 = dst.reshape(NW, NCH, CHUNK)
    cntp = _hist(src, dst)
    zu, zv = _fold(x, cntp, W0, b0, g0, bt0)
    sp = _pass1(zu, zv, src4, dst4)
    s = jnp.sum(sp, axis=0)
    mu1 = s[0] / N_EDGES
    var1 = jnp.maximum(s[1] / N_EDGES - mu1 * mu1, 0.0)
    a1 = g1 * lax.rsqrt(var1 + EPS)
    w = W1[0] * a1
    c_out = jnp.dot(bt1 - mu1 * a1, W1[0]) + b1[0]
    wc = jnp.stack([w, jnp.full((D,), c_out)])
    out = _pass2(zu, zv, src4, dst4, wc)
    return out.reshape(N_EDGES, 1)


# speedup vs baseline: 5.2095x; 1.1408x over previous
import functools

import jax
import jax.numpy as jnp
from jax import lax
from jax.experimental import pallas as pl
from jax.experimental.pallas import tpu as pltpu
from jax.experimental.pallas import tpu_sc as plsc

N_NODES = 10000
N_EDGES = 320000
D = 128
EPS = 1e-5

NC = 2
NS = 16
NW = NC * NS
EPW = N_EDGES // NW
CHUNK = 100
NCH = EPW // CHUNK
NV = D // 16

_mesh = plsc.VectorSubcoreMesh(core_axis_name="c", subcore_axis_name="s")


def _cols():
    lane = lax.iota(jnp.int32, 16)
    return [lane + 16 * f for f in range(NV)]


def _wid():
    return lax.axis_index("s") * NC + lax.axis_index("c")


def _hist_body(src_hbm, dst_hbm, out_hbm, idx_v, cnt_u, cnt_d):
    wid = _wid()
    base = wid * EPW
    zeros = jnp.zeros((16,), jnp.float32)

    def zero_body(i, _):
        cnt_u[pl.ds(i * 16, 16)] = zeros
        cnt_d[pl.ds(i * 16, 16)] = zeros
        return 0

    lax.fori_loop(0, N_NODES // 16, zero_body, 0)

    ones = jnp.ones((16,), jnp.float32)

    def count_into(cnt_ref):
        def body(i, _):
            ii = idx_v[pl.ds(i * 16, 16)]
            c, last = plsc.scan_count(ii)
            plsc.addupdate_scatter(cnt_ref, [ii], c.astype(jnp.float32),
                                   mask=last)
            return 0
        lax.fori_loop(0, EPW // 16, body, 0)

    pltpu.sync_copy(src_hbm.at[pl.ds(base, EPW)], idx_v)
    count_into(cnt_u)
    pltpu.sync_copy(dst_hbm.at[pl.ds(base, EPW)], idx_v)
    count_into(cnt_d)

    pltpu.sync_copy(cnt_u, out_hbm.at[wid, 0])
    pltpu.sync_copy(cnt_d, out_hbm.at[wid, 1])


_hist = pl.kernel(
    _hist_body,
    out_type=jax.ShapeDtypeStruct((NW, 2, N_NODES), jnp.float32),
    mesh=_mesh,
    compiler_params=pltpu.CompilerParams(needs_layout_passes=False),
    scratch_types=[
        pltpu.VMEM((EPW,), jnp.int32),
        pltpu.VMEM((N_NODES,), jnp.float32),
        pltpu.VMEM((N_NODES,), jnp.float32),
    ],
)


def _fold_body(x_ref, cntp_ref, w0_ref, b0_ref, g0_ref, bt0_ref,
               zu_ref, zv_ref):
    x = x_ref[...]
    cnt = jnp.sum(cntp_ref[...], axis=0)
    dn = (((1,), (0,)), ((), ()))
    s = lax.dot_general(cnt, x, dn, preferred_element_type=jnp.float32)
    q = lax.dot_general(cnt, x * x, dn, preferred_element_type=jnp.float32)
    mu = s / N_EDGES
    var = jnp.maximum(q / N_EDGES - mu * mu, 0.0)
    a0 = g0_ref[...].reshape(2, D) * lax.rsqrt(var + EPS)
    c0 = bt0_ref[...].reshape(2, D) - mu * a0
    w0 = w0_ref[...]
    w0u = w0[:, :D] * a0[0][None, :]
    w0v = w0[:, D:] * a0[1][None, :]
    dnv = (((1,), (0,)), ((), ()))
    b0eff = (b0_ref[...]
             + lax.dot_general(w0[:, :D], c0[0], dnv,
                               preferred_element_type=jnp.float32)
             + lax.dot_general(w0[:, D:], c0[1], dnv,
                               preferred_element_type=jnp.float32))
    dnt = (((1,), (1,)), ((), ()))
    zu_ref[...] = lax.dot_general(x, w0u, dnt,
                                  preferred_element_type=jnp.float32)
    zv_ref[...] = lax.dot_general(x, w0v, dnt,
                                  preferred_element_type=jnp.float32) \
        + b0eff[None, :]


_fold = pl.pallas_call(
    _fold_body,
    out_shape=(jax.ShapeDtypeStruct((N_NODES, D), jnp.float32),
               jax.ShapeDtypeStruct((N_NODES, D), jnp.float32)),
)


def _stage_idx(src4_hbm, dst4_hbm, idxs):
    wid = _wid()
    pltpu.sync_copy(src4_hbm.at[wid], idxs.at[0])
    pltpu.sync_copy(dst4_hbm.at[wid], idxs.at[1])


def _start(zu_hbm, zv_hbm, idxs, ubuf, vbuf, sems, b, ci):
    pltpu.async_copy(zu_hbm.at[idxs.at[0, ci]], ubuf.at[b], sems[0][b])
    pltpu.async_copy(zv_hbm.at[idxs.at[1, ci]], vbuf.at[b], sems[1][b])


def _wait(zu_hbm, zv_hbm, idxs, ubuf, vbuf, sems, b, ci):
    pltpu.make_async_copy(zu_hbm.at[idxs.at[0, ci]], ubuf.at[b],
                          sems[0][b]).wait()
    pltpu.make_async_copy(zv_hbm.at[idxs.at[1, ci]], vbuf.at[b],
                          sems[1][b]).wait()


def _pass1_body(zu_hbm, zv_hbm, src4_hbm, dst4_hbm, out_hbm,
                idxs, ubuf, vbuf, sbuf, su0, su1, sv0, sv1):
    wid = _wid()
    sems = ((su0, su1), (sv0, sv1))
    COLS = _cols()
    _stage_idx(src4_hbm, dst4_hbm, idxs)

    _start(zu_hbm, zv_hbm, idxs, ubuf, vbuf, sems, 0, 0)
    _start(zu_hbm, zv_hbm, idxs, ubuf, vbuf, sems, 1, 1)

    def make_inner(b):
        ub = ubuf.at[b]
        vb = vbuf.at[b]

        def edge_body(i, accs):
            out = list(accs)
            e0 = 2 * i
            row0 = jnp.full((16,), e0, jnp.int32)
            row1 = jnp.full((16,), e0 + 1, jnp.int32)
            for f in range(NV):
                u0 = plsc.load_gather(ub, [row0, COLS[f]])
                v0 = plsc.load_gather(vb, [row0, COLS[f]])
                u1 = plsc.load_gather(ub, [row1, COLS[f]])
                v1 = plsc.load_gather(vb, [row1, COLS[f]])
                r0 = jnp.maximum(u0 + v0, 0.0)
                r1 = jnp.maximum(u1 + v1, 0.0)
                out[f] = out[f] + (r0 + r1)
                out[NV + f] = out[NV + f] + (r0 * r0 + r1 * r1)
            return tuple(out)

        return edge_body

    inner = [make_inner(0), make_inner(1)]

    def outer(cio, accs):
        for b in range(2):
            ci = 2 * cio + b
            _wait(zu_hbm, zv_hbm, idxs, ubuf, vbuf, sems, b, ci)
            accs = lax.fori_loop(0, CHUNK // 2, inner[b], accs)
            _start(zu_hbm, zv_hbm, idxs, ubuf, vbuf, sems, b, ci + 2)
        return accs

    accs = tuple(jnp.zeros((16,), jnp.float32) for _ in range(2 * NV))
    accs = lax.fori_loop(0, NCH // 2 - 1, outer, accs)
    for b in range(2):
        ci = NCH - 2 + b
        _wait(zu_hbm, zv_hbm, idxs, ubuf, vbuf, sems, b, ci)
        accs = lax.fori_loop(0, CHUNK // 2, inner[b], accs)

    for f in range(NV):
        sbuf[0, pl.ds(16 * f, 16)] = accs[f]
        sbuf[1, pl.ds(16 * f, 16)] = accs[NV + f]
    pltpu.sync_copy(sbuf, out_hbm.at[wid])


_pass1 = pl.kernel(
    _pass1_body,
    out_type=jax.ShapeDtypeStruct((NW, 2, D), jnp.float32),
    mesh=_mesh,
    compiler_params=pltpu.CompilerParams(needs_layout_passes=False),
    scratch_types=[
        pltpu.VMEM((2, NCH, CHUNK), jnp.int32),
        pltpu.VMEM((2, CHUNK, D), jnp.float32),
        pltpu.VMEM((2, CHUNK, D), jnp.float32),
        pltpu.VMEM((2, D), jnp.float32),
        pltpu.SemaphoreType.DMA,
        pltpu.SemaphoreType.DMA,
        pltpu.SemaphoreType.DMA,
        pltpu.SemaphoreType.DMA,
    ],
)


def _pass2_body(zu_hbm, zv_hbm, src4_hbm, dst4_hbm, wc_hbm, out_hbm,
                idxs, ubuf, vbuf, wcv, obuf, su0, su1, sv0, sv1):
    wid = _wid()
    base = wid * EPW
    sems = ((su0, su1), (sv0, sv1))
    COLS = _cols()
    _stage_idx(src4_hbm, dst4_hbm, idxs)
    pltpu.sync_copy(wc_hbm, wcv)
    wv = [wcv[0, pl.ds(16 * f, 16)] for f in range(NV)]
    cvec = wcv[1, pl.ds(0, 16)]
    lane15 = lax.iota(jnp.int32, 16) == 15

    _start(zu_hbm, zv_hbm, idxs, ubuf, vbuf, sems, 0, 0)
    _start(zu_hbm, zv_hbm, idxs, ubuf, vbuf, sems, 1, 1)

    def make_inner(b):
        ub = ubuf.at[b]
        vb = vbuf.at[b]

        def edge_body(i, ci):
            e0 = 2 * i
            acc0 = None
            acc1 = None
            row0 = jnp.full((16,), e0, jnp.int32)
            row1 = jnp.full((16,), e0 + 1, jnp.int32)
            for f in range(NV):
                u0 = plsc.load_gather(ub, [row0, COLS[f]])
                v0 = plsc.load_gather(vb, [row0, COLS[f]])
                u1 = plsc.load_gather(ub, [row1, COLS[f]])
                v1 = plsc.load_gather(vb, [row1, COLS[f]])
                t0 = jnp.maximum(u0 + v0, 0.0) * wv[f]
                t1 = jnp.maximum(u1 + v1, 0.0) * wv[f]
                acc0 = t0 if acc0 is None else acc0 + t0
                acc1 = t1 if acc1 is None else acc1 + t1
            cs0 = plsc.cumsum(acc0) + cvec
            cs1 = plsc.cumsum(acc1) + cvec
            tgt0 = jnp.full((16,), ci * CHUNK + e0, jnp.int32)
            tgt1 = jnp.full((16,), ci * CHUNK + e0 + 1, jnp.int32)
            plsc.store_scatter(obuf, [tgt0], cs0, mask=lane15)
            plsc.store_scatter(obuf, [tgt1], cs1, mask=lane15)
            return ci

        return edge_body

    inner = [make_inner(0), make_inner(1)]

    def outer(cio, _):
        for b in range(2):
            ci = 2 * cio + b
            _wait(zu_hbm, zv_hbm, idxs, ubuf, vbuf, sems, b, ci)
            lax.fori_loop(0, CHUNK // 2, inner[b], ci)
            _start(zu_hbm, zv_hbm, idxs, ubuf, vbuf, sems, b, ci + 2)
        return 0

    lax.fori_loop(0, NCH // 2 - 1, outer, 0)
    for b in range(2):
        ci = NCH - 2 + b
        _wait(zu_hbm, zv_hbm, idxs, ubuf, vbuf, sems, b, ci)
        lax.fori_loop(0, CHUNK // 2, inner[b], ci)

    pltpu.sync_copy(obuf, out_hbm.at[pl.ds(base, EPW)])


_pass2 = pl.kernel(
    _pass2_body,
    out_type=jax.ShapeDtypeStruct((N_EDGES,), jnp.float32),
    mesh=_mesh,
    compiler_params=pltpu.CompilerParams(needs_layout_passes=False),
    scratch_types=[
        pltpu.VMEM((2, NCH, CHUNK), jnp.int32),
        pltpu.VMEM((2, CHUNK, D), jnp.float32),
        pltpu.VMEM((2, CHUNK, D), jnp.float32),
        pltpu.VMEM((2, D), jnp.float32),
        pltpu.VMEM((EPW,), jnp.float32),
        pltpu.SemaphoreType.DMA,
        pltpu.SemaphoreType.DMA,
        pltpu.SemaphoreType.DMA,
        pltpu.SemaphoreType.DMA,
    ],
)


def kernel(x, edge_index, W0, b0, W1, b1, g0, bt0, g1, bt1):
    src = edge_index[0]
    dst = edge_index[1]
    src4 = src.reshape(NW, NCH, CHUNK)
    dst4 = dst.reshape(NW, NCH, CHUNK)
    cntp = _hist(src, dst)
    zu, zv = _fold(x, cntp, W0, b0, g0, bt0)
    sp = _pass1(zu, zv, src4, dst4)
    s = jnp.sum(sp, axis=0)
    mu1 = s[0] / N_EDGES
    var1 = jnp.maximum(s[1] / N_EDGES - mu1 * mu1, 0.0)
    a1 = g1 * lax.rsqrt(var1 + EPS)
    w = W1[0] * a1
    c_out = jnp.dot(bt1 - mu1 * a1, W1[0]) + b1[0]
    wc = jnp.stack([w, jnp.full((D,), c_out)])
    out = _pass2(zu, zv, src4, dst4, wc)
    return out.reshape(N_EDGES, 1)


# speedup vs baseline: 5.2617x; 1.0100x over previous
import functools

import jax
import jax.numpy as jnp
from jax import lax
from jax.experimental import pallas as pl
from jax.experimental.pallas import tpu as pltpu
from jax.experimental.pallas import tpu_sc as plsc

N_NODES = 10000
N_EDGES = 320000
D = 128
EPS = 1e-5

NC = 2
NS = 16
NW = NC * NS
EPW = N_EDGES // NW
CHUNK = 100
NCH = EPW // CHUNK
NV = D // 16

_mesh = plsc.VectorSubcoreMesh(core_axis_name="c", subcore_axis_name="s")


def _cols():
    lane = lax.iota(jnp.int32, 16)
    return [lane + 16 * f for f in range(NV)]


def _wid():
    return lax.axis_index("s") * NC + lax.axis_index("c")


def _hist_body(src_hbm, dst_hbm, out_hbm, idx_v, cnt_u, cnt_d):
    wid = _wid()
    base = wid * EPW
    zeros = jnp.zeros((16,), jnp.float32)

    def zero_body(i, _):
        cnt_u[pl.ds(i * 16, 16)] = zeros
        cnt_d[pl.ds(i * 16, 16)] = zeros
        return 0

    lax.fori_loop(0, N_NODES // 16, zero_body, 0)

    ones = jnp.ones((16,), jnp.float32)

    def count_into(cnt_ref):
        def body(i, _):
            ii = idx_v[pl.ds(i * 16, 16)]
            c, last = plsc.scan_count(ii)
            plsc.addupdate_scatter(cnt_ref, [ii], c.astype(jnp.float32),
                                   mask=last)
            return 0
        lax.fori_loop(0, EPW // 16, body, 0)

    pltpu.sync_copy(src_hbm.at[pl.ds(base, EPW)], idx_v)
    count_into(cnt_u)
    pltpu.sync_copy(dst_hbm.at[pl.ds(base, EPW)], idx_v)
    count_into(cnt_d)

    pltpu.sync_copy(cnt_u, out_hbm.at[wid, 0])
    pltpu.sync_copy(cnt_d, out_hbm.at[wid, 1])


_hist = pl.kernel(
    _hist_body,
    out_type=jax.ShapeDtypeStruct((NW, 2, N_NODES), jnp.float32),
    mesh=_mesh,
    compiler_params=pltpu.CompilerParams(needs_layout_passes=False),
    scratch_types=[
        pltpu.VMEM((EPW,), jnp.int32),
        pltpu.VMEM((N_NODES,), jnp.float32),
        pltpu.VMEM((N_NODES,), jnp.float32),
    ],
)


def _fold_body(x_ref, cntp_ref, w0_ref, b0_ref, g0_ref, bt0_ref,
               zu_ref, zv_ref):
    x = x_ref[...]
    cnt = jnp.sum(cntp_ref[...], axis=0)
    dn = (((1,), (0,)), ((), ()))
    s = lax.dot_general(cnt, x, dn, preferred_element_type=jnp.float32)
    q = lax.dot_general(cnt, x * x, dn, preferred_element_type=jnp.float32)
    mu = s / N_EDGES
    var = jnp.maximum(q / N_EDGES - mu * mu, 0.0)
    a0 = g0_ref[...].reshape(2, D) * lax.rsqrt(var + EPS)
    c0 = bt0_ref[...].reshape(2, D) - mu * a0
    w0 = w0_ref[...]
    w0u = w0[:, :D] * a0[0][None, :]
    w0v = w0[:, D:] * a0[1][None, :]
    dnv = (((1,), (0,)), ((), ()))
    b0eff = (b0_ref[...]
             + lax.dot_general(w0[:, :D], c0[0], dnv,
                               preferred_element_type=jnp.float32)
             + lax.dot_general(w0[:, D:], c0[1], dnv,
                               preferred_element_type=jnp.float32))
    dnt = (((1,), (1,)), ((), ()))
    zu_ref[...] = lax.dot_general(x, w0u, dnt,
                                  preferred_element_type=jnp.float32)
    zv_ref[...] = lax.dot_general(x, w0v, dnt,
                                  preferred_element_type=jnp.float32) \
        + b0eff[None, :]


_fold = pl.pallas_call(
    _fold_body,
    out_shape=(jax.ShapeDtypeStruct((N_NODES, D), jnp.float32),
               jax.ShapeDtypeStruct((N_NODES, D), jnp.float32)),
)


def _stage_idx(src4_hbm, dst4_hbm, idxs):
    wid = _wid()
    pltpu.sync_copy(src4_hbm.at[wid], idxs.at[0])
    pltpu.sync_copy(dst4_hbm.at[wid], idxs.at[1])


def _start(zu_hbm, zv_hbm, idxs, ubuf, vbuf, sems, b, ci):
    pltpu.async_copy(zu_hbm.at[idxs.at[0, ci]], ubuf.at[b], sems[0][b])
    pltpu.async_copy(zv_hbm.at[idxs.at[1, ci]], vbuf.at[b], sems[1][b])


def _wait(zu_hbm, zv_hbm, idxs, ubuf, vbuf, sems, b, ci):
    pltpu.make_async_copy(zu_hbm.at[idxs.at[0, ci]], ubuf.at[b],
                          sems[0][b]).wait()
    pltpu.make_async_copy(zv_hbm.at[idxs.at[1, ci]], vbuf.at[b],
                          sems[1][b]).wait()


def _pass1_body(zu_hbm, zv_hbm, src4_hbm, dst4_hbm, out_hbm,
                idxs, ubuf, vbuf, sbuf, su0, su1, sv0, sv1):
    wid = _wid()
    sems = ((su0, su1), (sv0, sv1))
    COLS = _cols()
    _stage_idx(src4_hbm, dst4_hbm, idxs)

    _start(zu_hbm, zv_hbm, idxs, ubuf, vbuf, sems, 0, 0)
    _start(zu_hbm, zv_hbm, idxs, ubuf, vbuf, sems, 1, 1)

    def make_inner(b):
        ub = ubuf.at[b]
        vb = vbuf.at[b]

        def edge_body(i, accs):
            out = list(accs)
            e0 = 2 * i
            row0 = jnp.full((16,), e0, jnp.int32)
            row1 = jnp.full((16,), e0 + 1, jnp.int32)
            for f in range(NV):
                u0 = plsc.load_gather(ub, [row0, COLS[f]])
                v0 = plsc.load_gather(vb, [row0, COLS[f]])
                u1 = plsc.load_gather(ub, [row1, COLS[f]])
                v1 = plsc.load_gather(vb, [row1, COLS[f]])
                r0 = jnp.maximum(u0 + v0, 0.0)
                r1 = jnp.maximum(u1 + v1, 0.0)
                out[f] = out[f] + (r0 + r1)
                out[NV + f] = out[NV + f] + (r0 * r0 + r1 * r1)
            return tuple(out)

        return edge_body

    inner = [make_inner(0), make_inner(1)]

    def outer(cio, accs):
        for b in range(2):
            ci = 2 * cio + b
            _wait(zu_hbm, zv_hbm, idxs, ubuf, vbuf, sems, b, ci)
            accs = lax.fori_loop(0, CHUNK // 2, inner[b], accs)
            _start(zu_hbm, zv_hbm, idxs, ubuf, vbuf, sems, b, ci + 2)
        return accs

    accs = tuple(jnp.zeros((16,), jnp.float32) for _ in range(2 * NV))
    accs = lax.fori_loop(0, NCH // 2 - 1, outer, accs)
    for b in range(2):
        ci = NCH - 2 + b
        _wait(zu_hbm, zv_hbm, idxs, ubuf, vbuf, sems, b, ci)
        accs = lax.fori_loop(0, CHUNK // 2, inner[b], accs)

    for f in range(NV):
        sbuf[0, pl.ds(16 * f, 16)] = accs[f]
        sbuf[1, pl.ds(16 * f, 16)] = accs[NV + f]
    pltpu.sync_copy(sbuf, out_hbm.at[wid])


_pass1 = pl.kernel(
    _pass1_body,
    out_type=jax.ShapeDtypeStruct((NW, 2, D), jnp.float32),
    mesh=_mesh,
    compiler_params=pltpu.CompilerParams(needs_layout_passes=False),
    scratch_types=[
        pltpu.VMEM((2, NCH, CHUNK), jnp.int32),
        pltpu.VMEM((2, CHUNK, D), jnp.float32),
        pltpu.VMEM((2, CHUNK, D), jnp.float32),
        pltpu.VMEM((2, D), jnp.float32),
        pltpu.SemaphoreType.DMA,
        pltpu.SemaphoreType.DMA,
        pltpu.SemaphoreType.DMA,
        pltpu.SemaphoreType.DMA,
    ],
)


def _pass2_body(zu_hbm, zv_hbm, src4_hbm, dst4_hbm, wc_hbm, out_hbm,
                idxs, ubuf, vbuf, wcv, obuf, su0, su1, sv0, sv1):
    wid = _wid()
    base = wid * EPW
    sems = ((su0, su1), (sv0, sv1))
    COLS = _cols()
    _stage_idx(src4_hbm, dst4_hbm, idxs)
    pltpu.sync_copy(wc_hbm, wcv)
    wv = [wcv[0, pl.ds(16 * f, 16)] for f in range(NV)]
    cvec = wcv[1, pl.ds(0, 16)]
    lane15 = lax.iota(jnp.int32, 16) == 15

    _start(zu_hbm, zv_hbm, idxs, ubuf, vbuf, sems, 0, 0)
    _start(zu_hbm, zv_hbm, idxs, ubuf, vbuf, sems, 1, 1)

    def make_inner(b):
        ub = ubuf.at[b]
        vb = vbuf.at[b]

        def edge_body(i, ci):
            e0 = 4 * i
            rows = [jnp.full((16,), e0 + k, jnp.int32) for k in range(4)]
            accs = [None, None, None, None]
            for f in range(NV):
                for k in range(4):
                    u = plsc.load_gather(ub, [rows[k], COLS[f]])
                    v = plsc.load_gather(vb, [rows[k], COLS[f]])
                    t = jnp.maximum(u + v, 0.0) * wv[f]
                    accs[k] = t if accs[k] is None else accs[k] + t
            for k in range(4):
                cs = plsc.cumsum(accs[k]) + cvec
                tgt = jnp.full((16,), ci * CHUNK + e0 + k, jnp.int32)
                plsc.store_scatter(obuf, [tgt], cs, mask=lane15)
            return ci
        return edge_body

    inner = [make_inner(0), make_inner(1)]

    def outer(cio, _):
        for b in range(2):
            ci = 2 * cio + b
            _wait(zu_hbm, zv_hbm, idxs, ubuf, vbuf, sems, b, ci)
            lax.fori_loop(0, CHUNK // 4, inner[b], ci)
            _start(zu_hbm, zv_hbm, idxs, ubuf, vbuf, sems, b, ci + 2)
        return 0

    lax.fori_loop(0, NCH // 2 - 1, outer, 0)
    for b in range(2):
        ci = NCH - 2 + b
        _wait(zu_hbm, zv_hbm, idxs, ubuf, vbuf, sems, b, ci)
        lax.fori_loop(0, CHUNK // 4, inner[b], ci)

    pltpu.sync_copy(obuf, out_hbm.at[pl.ds(base, EPW)])


_pass2 = pl.kernel(
    _pass2_body,
    out_type=jax.ShapeDtypeStruct((N_EDGES,), jnp.float32),
    mesh=_mesh,
    compiler_params=pltpu.CompilerParams(needs_layout_passes=False),
    scratch_types=[
        pltpu.VMEM((2, NCH, CHUNK), jnp.int32),
        pltpu.VMEM((2, CHUNK, D), jnp.float32),
        pltpu.VMEM((2, CHUNK, D), jnp.float32),
        pltpu.VMEM((2, D), jnp.float32),
        pltpu.VMEM((EPW,), jnp.float32),
        pltpu.SemaphoreType.DMA,
        pltpu.SemaphoreType.DMA,
        pltpu.SemaphoreType.DMA,
        pltpu.SemaphoreType.DMA,
    ],
)


def kernel(x, edge_index, W0, b0, W1, b1, g0, bt0, g1, bt1):
    src = edge_index[0]
    dst = edge_index[1]
    src4 = src.reshape(NW, NCH, CHUNK)
    dst4 = dst.reshape(NW, NCH, CHUNK)
    cntp = _hist(src, dst)
    zu, zv = _fold(x, cntp, W0, b0, g0, bt0)
    sp = _pass1(zu, zv, src4, dst4)
    s = jnp.sum(sp, axis=0)
    mu1 = s[0] / N_EDGES
    var1 = jnp.maximum(s[1] / N_EDGES - mu1 * mu1, 0.0)
    a1 = g1 * lax.rsqrt(var1 + EPS)
    w = W1[0] * a1
    c_out = jnp.dot(bt1 - mu1 * a1, W1[0]) + b1[0]
    wc = jnp.stack([w, jnp.full((D,), c_out)])
    out = _pass2(zu, zv, src4, dst4, wc)
    return out.reshape(N_EDGES, 1)
